# initial kernel scaffold (unmeasured)
import jax
import jax.numpy as jnp
from jax import lax
from jax.experimental import pallas as pl
from jax.experimental.pallas import tpu as pltpu

T = 4096
D = 2048
F = 4096
E_LOC = 4
C = 1280
BF = 512
NF = F // BF

_MESH = pl.DeviceIdType.MESH


def _xneighbor():
    return (1 - lax.axis_index("x"), lax.axis_index("y"))


def _partner_barrier(other):
    barrier = pltpu.get_barrier_semaphore()
    pl.semaphore_signal(barrier, inc=1, device_id=other, device_id_type=_MESH)
    pl.semaphore_wait(barrier, 1)


def _exchange2_body(xb_ref, as_ref, xb_out, as_out, sx, sa, rx, ra):
    other = _xneighbor()
    _partner_barrier(other)
    cx = pltpu.make_async_remote_copy(
        src_ref=xb_ref, dst_ref=xb_out, send_sem=sx, recv_sem=rx,
        device_id=other, device_id_type=_MESH)
    ca = pltpu.make_async_remote_copy(
        src_ref=as_ref, dst_ref=as_out, send_sem=sa, recv_sem=ra,
        device_id=other, device_id_type=_MESH)
    cx.start()
    ca.start()
    cx.wait()
    ca.wait()


def _exchange1_body(y_ref, y_out, s, r):
    other = _xneighbor()
    _partner_barrier(other)
    c = pltpu.make_async_remote_copy(
        src_ref=y_ref, dst_ref=y_out, send_sem=s, recv_sem=r,
        device_id=other, device_id_type=_MESH)
    c.start()
    c.wait()


def _moe_body(xb_ref, w1_ref, w2_ref, y_ref):
    f = pl.program_id(1)

    @pl.when(f == 0)
    def _():
        y_ref[...] = jnp.zeros_like(y_ref)

    h = jnp.dot(xb_ref[...], w1_ref[...].astype(jnp.bfloat16),
                preferred_element_type=jnp.float32)
    h = jnp.maximum(h, 0.0).astype(jnp.bfloat16)
    y_ref[...] += jnp.dot(h, w2_ref[...].astype(jnp.bfloat16),
                          preferred_element_type=jnp.float32)


def kernel(x, assign, W1, W2):
    my_x = lax.axis_index("x")

    xb = x.astype(jnp.bfloat16)
    assign2d = assign.reshape(32, 128)
    xb_other, as_other = pl.pallas_call(
        _exchange2_body,
        out_shape=[
            jax.ShapeDtypeStruct((T, D), jnp.bfloat16),
            jax.ShapeDtypeStruct((32, 128), jnp.int32),
        ],
        in_specs=[pl.BlockSpec(memory_space=pltpu.VMEM)] * 2,
        out_specs=[pl.BlockSpec(memory_space=pltpu.VMEM)] * 2,
        scratch_shapes=[pltpu.SemaphoreType.DMA] * 4,
        compiler_params=pltpu.CompilerParams(collective_id=0),
    )(xb, assign2d)
    assign_other = as_other.reshape(T)

    x_all = jnp.concatenate([xb, xb_other], axis=0)
    assign_all = jnp.concatenate([assign, assign_other])
    local_e = assign_all - E_LOC * my_x
    key = jnp.where((local_e >= 0) & (local_e < E_LOC), local_e, E_LOC)
    sort_idx = jnp.argsort(key, stable=True).astype(jnp.int32)
    key_sorted = key[sort_idx]
    counts = jnp.bincount(key, length=E_LOC + 1)
    ex_cumsum = jnp.concatenate(
        [jnp.zeros((1,), counts.dtype), jnp.cumsum(counts)[:-1]])
    rank = jnp.arange(2 * T, dtype=jnp.int32) - ex_cumsum[key_sorted]
    slot = jnp.where((key_sorted < E_LOC) & (rank < C),
                     key_sorted * C + rank, E_LOC * C).astype(jnp.int32)
    slot_src = jnp.zeros((E_LOC * C + 1,), jnp.int32).at[slot].set(sort_idx)
    Xbuf = jnp.take(x_all, slot_src[:E_LOC * C], axis=0).reshape(E_LOC, C, D)
    slot_of_token = jnp.zeros((2 * T,), jnp.int32).at[sort_idx].set(slot)

    Y = pl.pallas_call(
        _moe_body,
        grid=(E_LOC, NF),
        out_shape=jax.ShapeDtypeStruct((E_LOC, C, D), jnp.float32),
        in_specs=[
            pl.BlockSpec((None, C, D), lambda e, f: (e, 0, 0)),
            pl.BlockSpec((None, D, BF), lambda e, f: (e, 0, f)),
            pl.BlockSpec((None, BF, D), lambda e, f: (e, f, 0)),
        ],
        out_specs=pl.BlockSpec((None, C, D), lambda e, f: (e, 0, 0)),
    )(Xbuf, W1, W2)

    Y_pad = jnp.concatenate(
        [Y.reshape(E_LOC * C, D), jnp.zeros((1, D), jnp.float32)], axis=0)
    out_all = jnp.take(Y_pad, slot_of_token, axis=0)
    mine = out_all[:T]
    theirs_bf = out_all[T:].astype(jnp.bfloat16)

    recv_bf = pl.pallas_call(
        _exchange1_body,
        out_shape=jax.ShapeDtypeStruct((T, D), jnp.bfloat16),
        in_specs=[pl.BlockSpec(memory_space=pltpu.VMEM)],
        out_specs=pl.BlockSpec(memory_space=pltpu.VMEM),
        scratch_shapes=[pltpu.SemaphoreType.DMA] * 2,
        compiler_params=pltpu.CompilerParams(collective_id=1),
    )(theirs_bf)

    return mine + recv_bf.astype(jnp.float32)


# baseline (device time: 7805373 ns/iter reference)
import jax
import jax.numpy as jnp
from jax import lax
from jax.experimental import pallas as pl
from jax.experimental.pallas import tpu as pltpu

T = 4096
D = 2048
F = 4096
E_LOC = 4
C = 1280
BF = 512
NF = F // BF

_MESH = pl.DeviceIdType.MESH


def _xneighbor():
    return (1 - lax.axis_index("x"), lax.axis_index("y"))


def _partner_barrier(other):
    barrier = pltpu.get_barrier_semaphore()
    pl.semaphore_signal(barrier, inc=1, device_id=other, device_id_type=_MESH)
    pl.semaphore_wait(barrier, 1)


def _exchange2_body(xb_ref, as_ref, xb_out, as_out, sx, sa, rx, ra):
    other = _xneighbor()
    _partner_barrier(other)
    cx = pltpu.make_async_remote_copy(
        src_ref=xb_ref, dst_ref=xb_out, send_sem=sx, recv_sem=rx,
        device_id=other, device_id_type=_MESH)
    ca = pltpu.make_async_remote_copy(
        src_ref=as_ref, dst_ref=as_out, send_sem=sa, recv_sem=ra,
        device_id=other, device_id_type=_MESH)
    cx.start()
    ca.start()
    cx.wait()
    ca.wait()


def _exchange1_body(y_ref, y_out, s, r):
    other = _xneighbor()
    _partner_barrier(other)
    c = pltpu.make_async_remote_copy(
        src_ref=y_ref, dst_ref=y_out, send_sem=s, recv_sem=r,
        device_id=other, device_id_type=_MESH)
    c.start()
    c.wait()


def _moe_body(xb_ref, w1_ref, w2_ref, y_ref):
    f = pl.program_id(1)

    @pl.when(f == 0)
    def _():
        y_ref[...] = jnp.zeros_like(y_ref)

    h = jnp.dot(xb_ref[...], w1_ref[...].astype(jnp.bfloat16),
                preferred_element_type=jnp.float32)
    h = jnp.maximum(h, 0.0).astype(jnp.bfloat16)
    y_ref[...] += jnp.dot(h, w2_ref[...].astype(jnp.bfloat16),
                          preferred_element_type=jnp.float32)


def kernel(x, assign, W1, W2):
    my_x = lax.axis_index("x")

    xb = x.astype(jnp.bfloat16)
    assign2d = assign.reshape(32, 128)
    xb_other, as_other = pl.pallas_call(
        _exchange2_body,
        out_shape=[
            jax.ShapeDtypeStruct((T, D), jnp.bfloat16),
            jax.ShapeDtypeStruct((32, 128), jnp.int32),
        ],
        in_specs=[pl.BlockSpec(memory_space=pltpu.VMEM)] * 2,
        out_specs=[pl.BlockSpec(memory_space=pltpu.VMEM)] * 2,
        scratch_shapes=[pltpu.SemaphoreType.DMA] * 4,
        compiler_params=pltpu.CompilerParams(collective_id=0),
    )(xb, assign2d)
    assign_other = as_other.reshape(T)

    x_all = jnp.concatenate([xb, xb_other], axis=0)
    assign_all = jnp.concatenate([assign, assign_other])
    local_e = assign_all - E_LOC * my_x
    key = jnp.where((local_e >= 0) & (local_e < E_LOC), local_e, E_LOC)
    sort_idx = jnp.argsort(key, stable=True).astype(jnp.int32)
    key_sorted = key[sort_idx]
    counts = jnp.bincount(key, length=E_LOC + 1)
    ex_cumsum = jnp.concatenate(
        [jnp.zeros((1,), counts.dtype), jnp.cumsum(counts)[:-1]])
    rank = jnp.arange(2 * T, dtype=jnp.int32) - ex_cumsum[key_sorted]
    slot = jnp.where((key_sorted < E_LOC) & (rank < C),
                     key_sorted * C + rank, E_LOC * C).astype(jnp.int32)
    slot_src = jnp.zeros((E_LOC * C + 1,), jnp.int32).at[slot].set(sort_idx)
    Xbuf = jnp.take(x_all, slot_src[:E_LOC * C], axis=0).reshape(E_LOC, C, D)
    slot_of_token = jnp.zeros((2 * T,), jnp.int32).at[sort_idx].set(slot)

    Y = pl.pallas_call(
        _moe_body,
        grid=(E_LOC, NF),
        out_shape=jax.ShapeDtypeStruct((E_LOC, C, D), jnp.float32),
        in_specs=[
            pl.BlockSpec((None, C, D), lambda e, f: (e, 0, 0)),
            pl.BlockSpec((None, D, BF), lambda e, f: (e, 0, f)),
            pl.BlockSpec((None, BF, D), lambda e, f: (e, f, 0)),
        ],
        out_specs=pl.BlockSpec((None, C, D), lambda e, f: (e, 0, 0)),
        compiler_params=pltpu.CompilerParams(
            vmem_limit_bytes=60 * 1024 * 1024),
    )(Xbuf, W1, W2)

    Y_pad = jnp.concatenate(
        [Y.reshape(E_LOC * C, D), jnp.zeros((1, D), jnp.float32)], axis=0)
    out_all = jnp.take(Y_pad, slot_of_token, axis=0)
    mine = out_all[:T]
    theirs_bf = out_all[T:].astype(jnp.bfloat16)

    recv_bf = pl.pallas_call(
        _exchange1_body,
        out_shape=jax.ShapeDtypeStruct((T, D), jnp.bfloat16),
        in_specs=[pl.BlockSpec(memory_space=pltpu.VMEM)],
        out_specs=pl.BlockSpec(memory_space=pltpu.VMEM),
        scratch_shapes=[pltpu.SemaphoreType.DMA] * 2,
        compiler_params=pltpu.CompilerParams(collective_id=1),
    )(theirs_bf)

    return mine + recv_bf.astype(jnp.float32)


# device time: 1138524 ns/iter; 6.8557x vs baseline; 6.8557x over previous
import jax
import jax.numpy as jnp
from jax import lax
from jax.experimental import pallas as pl
from jax.experimental.pallas import tpu as pltpu

T = 4096
D = 2048
F = 4096
E_LOC = 4
C = 1280
S = E_LOC * C

BF = 512
NF = F // BF
BM1, BK1 = 640, 2048
BM2, BK2 = 1024, 1280

_MESH = pl.DeviceIdType.MESH
_VMEM_LIM = pltpu.CompilerParams(vmem_limit_bytes=60 * 1024 * 1024)


def _xneighbor():
    return (1 - lax.axis_index("x"), lax.axis_index("y"))


def _partner_barrier(other):
    barrier = pltpu.get_barrier_semaphore()
    pl.semaphore_signal(barrier, inc=1, device_id=other, device_id_type=_MESH)
    pl.semaphore_wait(barrier, 1)


def _exchA_body(xb_ref, as_ref, xall_out, asall_out, sx, sa, rx, ra):
    other = _xneighbor()
    _partner_barrier(other)
    cx = pltpu.make_async_remote_copy(
        src_ref=xb_ref, dst_ref=xall_out.at[pl.ds(T, T), :],
        send_sem=sx, recv_sem=rx, device_id=other, device_id_type=_MESH)
    ca = pltpu.make_async_remote_copy(
        src_ref=as_ref, dst_ref=asall_out.at[pl.ds(32, 32), :],
        send_sem=sa, recv_sem=ra, device_id=other, device_id_type=_MESH)
    cx.start()
    ca.start()
    xall_out[:T, :] = xb_ref[...]
    asall_out[:32, :] = as_ref[...]
    cx.wait()
    ca.wait()


def _exchE_body(y_ref, y_out, s, r):
    other = _xneighbor()
    _partner_barrier(other)
    c = pltpu.make_async_remote_copy(
        src_ref=y_ref, dst_ref=y_out, send_sem=s, recv_sem=r,
        device_id=other, device_id_type=_MESH)
    c.start()
    c.wait()


def _p1_body(idx_ref, x_ref, out_ref, acc):
    k = pl.program_id(1)

    @pl.when(k == 0)
    def _():
        acc[...] = jnp.zeros_like(acc)

    iota = jax.lax.broadcasted_iota(jnp.int32, (BM1, BK1), 1) + k * BK1
    oh = (idx_ref[...] == iota).astype(jnp.bfloat16)
    acc[...] += jnp.dot(oh, x_ref[...], preferred_element_type=jnp.float32)

    @pl.when(k == pl.num_programs(1) - 1)
    def _():
        out_ref[...] = acc[...].astype(jnp.bfloat16)


def _p2_body(idx_ref, y_ref, out_ref, acc):
    k = pl.program_id(1)

    @pl.when(k == 0)
    def _():
        acc[...] = jnp.zeros_like(acc)

    iota = jax.lax.broadcasted_iota(jnp.int32, (BM2, BK2), 1) + k * BK2
    oh = (idx_ref[...] == iota).astype(jnp.bfloat16)
    acc[...] += jnp.dot(oh, y_ref[...], preferred_element_type=jnp.float32)

    @pl.when(k == pl.num_programs(1) - 1)
    def _():
        out_ref[...] = acc[...].astype(jnp.bfloat16)


def _moe_body(xb_ref, w1_ref, w2_ref, y_ref, acc):
    f = pl.program_id(1)

    @pl.when(f == 0)
    def _():
        acc[...] = jnp.zeros_like(acc)

    h = jnp.dot(xb_ref[...], w1_ref[...].astype(jnp.bfloat16),
                preferred_element_type=jnp.float32)
    h = jnp.maximum(h, 0.0).astype(jnp.bfloat16)
    acc[...] += jnp.dot(h, w2_ref[...].astype(jnp.bfloat16),
                        preferred_element_type=jnp.float32)

    @pl.when(f == NF - 1)
    def _():
        y_ref[...] = acc[...].astype(jnp.bfloat16)


def kernel(x, assign, W1, W2):
    my_x = lax.axis_index("x")

    xb = x.astype(jnp.bfloat16)
    assign2d = assign.reshape(32, 128)
    x_all, as_all = pl.pallas_call(
        _exchA_body,
        out_shape=[
            jax.ShapeDtypeStruct((2 * T, D), jnp.bfloat16),
            jax.ShapeDtypeStruct((64, 128), jnp.int32),
        ],
        in_specs=[pl.BlockSpec(memory_space=pltpu.VMEM)] * 2,
        out_specs=[pl.BlockSpec(memory_space=pltpu.VMEM)] * 2,
        scratch_shapes=[pltpu.SemaphoreType.DMA] * 4,
        compiler_params=pltpu.CompilerParams(
            collective_id=0, vmem_limit_bytes=60 * 1024 * 1024),
    )(xb, assign2d)
    assign_all = as_all.reshape(2 * T)

    local_e = assign_all - E_LOC * my_x
    key = jnp.where((local_e >= 0) & (local_e < E_LOC), local_e, E_LOC)
    sort_idx = jnp.argsort(key, stable=True).astype(jnp.int32)
    key_sorted = key[sort_idx]
    counts = jnp.bincount(key, length=E_LOC + 1)
    ex_cumsum = jnp.concatenate(
        [jnp.zeros((1,), counts.dtype), jnp.cumsum(counts)[:-1]])
    rank = jnp.arange(2 * T, dtype=jnp.int32) - ex_cumsum[key_sorted]
    slot = jnp.where((key_sorted < E_LOC) & (rank < C),
                     key_sorted * C + rank, S).astype(jnp.int32)
    slot_src = jnp.full((S + 1,), 2 * T, jnp.int32).at[slot].set(sort_idx)
    slot_src2d = slot_src[:S, None]
    slot_of_token = jnp.zeros((2 * T,), jnp.int32).at[sort_idx].set(slot)
    tok_slot2d = slot_of_token[:, None]

    Xbuf = pl.pallas_call(
        _p1_body,
        grid=(S // BM1, 2 * T // BK1),
        in_specs=[
            pl.BlockSpec((BM1, 1), lambda m, k: (m, 0)),
            pl.BlockSpec((BK1, D), lambda m, k: (k, 0)),
        ],
        out_specs=pl.BlockSpec((BM1, D), lambda m, k: (m, 0)),
        out_shape=jax.ShapeDtypeStruct((S, D), jnp.bfloat16),
        scratch_shapes=[pltpu.VMEM((BM1, D), jnp.float32)],
        compiler_params=_VMEM_LIM,
    )(slot_src2d, x_all)

    Y = pl.pallas_call(
        _moe_body,
        grid=(E_LOC, NF),
        in_specs=[
            pl.BlockSpec((None, C, D), lambda e, f: (e, 0, 0)),
            pl.BlockSpec((None, D, BF), lambda e, f: (e, 0, f)),
            pl.BlockSpec((None, BF, D), lambda e, f: (e, f, 0)),
        ],
        out_specs=pl.BlockSpec((None, C, D), lambda e, f: (e, 0, 0)),
        out_shape=jax.ShapeDtypeStruct((E_LOC, C, D), jnp.bfloat16),
        scratch_shapes=[pltpu.VMEM((C, D), jnp.float32)],
        compiler_params=_VMEM_LIM,
    )(Xbuf.reshape(E_LOC, C, D), W1, W2)

    out_all = pl.pallas_call(
        _p2_body,
        grid=(2 * T // BM2, S // BK2),
        in_specs=[
            pl.BlockSpec((BM2, 1), lambda m, k: (m, 0)),
            pl.BlockSpec((BK2, D), lambda m, k: (k, 0)),
        ],
        out_specs=pl.BlockSpec((BM2, D), lambda m, k: (m, 0)),
        out_shape=jax.ShapeDtypeStruct((2 * T, D), jnp.bfloat16),
        scratch_shapes=[pltpu.VMEM((BM2, D), jnp.float32)],
        compiler_params=_VMEM_LIM,
    )(tok_slot2d, Y.reshape(S, D))

    recv = pl.pallas_call(
        _exchE_body,
        out_shape=jax.ShapeDtypeStruct((T, D), jnp.bfloat16),
        in_specs=[pl.BlockSpec(memory_space=pltpu.VMEM)],
        out_specs=pl.BlockSpec(memory_space=pltpu.VMEM),
        scratch_shapes=[pltpu.SemaphoreType.DMA] * 2,
        compiler_params=pltpu.CompilerParams(
            collective_id=1, vmem_limit_bytes=60 * 1024 * 1024),
    )(out_all[T:])

    return out_all[:T].astype(jnp.float32) + recv.astype(jnp.float32)
